# baseline (speedup 1.0000x reference)
import functools

import jax
import jax.numpy as jnp
from jax import lax
from jax.experimental import pallas as pl
from jax.experimental.pallas import tpu as pltpu
from jax.experimental.pallas import tpu_sc as plsc

H = 128
N = 10000
E = 160000
EW = 80000

BN = 1000
BE = 1000

_SC_INFO = plsc.get_sparse_core_info()
NC = _SC_INFO.num_cores
NS = _SC_INFO.num_subcores
NW = NC * NS
GCH = 768
SCH = 128

NP = 10240
_RPT = NP // 16


def _sc_gather(table, idx2d, n_rows):
    nchunks = n_rows // GCH
    mesh = plsc.VectorSubcoreMesh(core_axis_name="c", subcore_axis_name="s")

    @functools.partial(
        pl.kernel, mesh=mesh,
        out_type=jax.ShapeDtypeStruct((n_rows, H), jnp.float32),
        scratch_types=[
            pltpu.VMEM((GCH,), jnp.int32),
            pltpu.VMEM((GCH, H), jnp.float32),
            pltpu.SemaphoreType.DMA,
        ],
    )
    def k(table_hbm, idx_hbm, out_hbm, idx_v, rows_v, sem):
        wid = lax.axis_index("s") * NC + lax.axis_index("c")
        nt = (nchunks - wid + NW - 1) // NW

        def body(i, _):
            j = wid + i * NW
            pltpu.sync_copy(idx_hbm.at[j], idx_v)
            pltpu.async_copy(table_hbm.at[idx_v], rows_v, sem).wait()
            pltpu.sync_copy(rows_v, out_hbm.at[pl.ds(j * GCH, GCH)])
            return 0

        lax.fori_loop(0, nt, body, 0)

    return k(table, idx2d)


def _sc_segment_sum(attr, idx2d, zeros):
    nchunks = idx2d.shape[0]
    mesh = plsc.VectorSubcoreMesh(core_axis_name="c", subcore_axis_name="s")

    @functools.partial(
        pl.kernel, mesh=mesh,
        out_type=jax.ShapeDtypeStruct((NC, NP, H), jnp.float32),
        scratch_types=[
            pltpu.VMEM_SHARED((NP, H), jnp.float32),
            pltpu.VMEM((SCH,), jnp.int32),
            pltpu.VMEM((SCH, H), jnp.float32),
        ],
    )
    def k(attr_hbm, idx_hbm, zeros_hbm, out_hbm, acc, idx_v, rows_v):
        c = lax.axis_index("c")
        s = lax.axis_index("s")
        wid = s * NC + c
        pltpu.sync_copy(zeros_hbm.at[pl.ds(s * _RPT, _RPT)],
                        acc.at[pl.ds(s * _RPT, _RPT)])
        plsc.subcore_barrier()
        nt = (nchunks - wid + NW - 1) // NW

        def body(i, _):
            j = wid + i * NW
            pltpu.sync_copy(idx_hbm.at[j], idx_v)
            pltpu.sync_copy(attr_hbm.at[pl.ds(j * SCH, SCH)], rows_v)
            pltpu.sync_copy(rows_v, acc.at[idx_v], add=True)
            return 0

        lax.fori_loop(0, nt, body, 0)
        plsc.subcore_barrier()
        for kk in range(_RPT // SCH):
            off = s * _RPT + kk * SCH
            pltpu.sync_copy(acc.at[pl.ds(off, SCH)], rows_v)
            pltpu.sync_copy(rows_v, out_hbm.at[c, pl.ds(off, SCH)])

    return k(attr, idx2d, zeros)


def _proj_body(x_ref, w_ref, out_ref):
    xb = x_ref[...]
    for k in range(5):
        out_ref[k] = jnp.dot(xb, w_ref[:, k * H:(k + 1) * H],
                             preferred_element_type=jnp.float32)


def _proj(x, wcat):
    return pl.pallas_call(
        _proj_body,
        grid=(N // BN,),
        in_specs=[
            pl.BlockSpec((BN, H), lambda i: (i, 0)),
            pl.BlockSpec((H, 5 * H), lambda i: (0, 0)),
        ],
        out_specs=pl.BlockSpec((5, BN, H), lambda i: (0, i, 0)),
        out_shape=jax.ShapeDtypeStruct((5, N, H), jnp.float32),
    )(x, wcat)


def _edge_mlp_body(gs_ref, gd_ref, ea_ref, w1c_ref, b1_ref, w2_ref, b2_ref,
                   g_ref, bln_ref, enew_ref, eout_ref):
    ea = ea_ref[...]
    pre = (gs_ref[...] + gd_ref[...] + b1_ref[...]
           + jnp.dot(ea, w1c_ref[...], preferred_element_type=jnp.float32))
    h = jnp.maximum(pre, 0.0)
    z = jnp.dot(h, w2_ref[...], preferred_element_type=jnp.float32) + b2_ref[...]
    mu = jnp.mean(z, axis=-1, keepdims=True)
    var = jnp.mean((z - mu) ** 2, axis=-1, keepdims=True)
    e_new = (z - mu) * jax.lax.rsqrt(var + 1e-5) * g_ref[...] + bln_ref[...]
    enew_ref[...] = e_new
    eout_ref[...] = ea + e_new


def _edge_mlp(g_all, attr, row0_src, row0_dst, w1c, b1, w2, b2, gg, bln, n_rows):
    wspec = pl.BlockSpec((H, H), lambda i: (0, 0))
    bspec = pl.BlockSpec((1, H), lambda i: (0, 0))
    return pl.pallas_call(
        _edge_mlp_body,
        grid=(n_rows // BE,),
        in_specs=[
            pl.BlockSpec((BE, H), lambda i, r=row0_src // BE: (r + i, 0)),
            pl.BlockSpec((BE, H), lambda i, r=row0_dst // BE: (r + i, 0)),
            pl.BlockSpec((BE, H), lambda i: (i, 0)),
            wspec, bspec, wspec, bspec, bspec, bspec,
        ],
        out_specs=[
            pl.BlockSpec((BE, H), lambda i: (i, 0)),
            pl.BlockSpec((BE, H), lambda i: (i, 0)),
        ],
        out_shape=[
            jax.ShapeDtypeStruct((n_rows, H), jnp.float32),
            jax.ShapeDtypeStruct((n_rows, H), jnp.float32),
        ],
    )(g_all, g_all, attr, w1c, b1.reshape(1, H), w2, b2.reshape(1, H),
      gg.reshape(1, H), bln.reshape(1, H))


def _node_mlp_body(x_ref, t_ref, ae0_ref, ae1_ref, aw0_ref, aw1_ref,
                   w1b_ref, w1c_ref, b1_ref, w2_ref, b2_ref, g_ref, bln_ref,
                   out_ref):
    ae = ae0_ref[0] + ae1_ref[0]
    aw = aw0_ref[0] + aw1_ref[0]
    pre = (t_ref[0] + b1_ref[...]
           + jnp.dot(ae, w1b_ref[...], preferred_element_type=jnp.float32)
           + jnp.dot(aw, w1c_ref[...], preferred_element_type=jnp.float32))
    h = jnp.maximum(pre, 0.0)
    z = jnp.dot(h, w2_ref[...], preferred_element_type=jnp.float32) + b2_ref[...]
    mu = jnp.mean(z, axis=-1, keepdims=True)
    var = jnp.mean((z - mu) ** 2, axis=-1, keepdims=True)
    x_new = (z - mu) * jax.lax.rsqrt(var + 1e-5) * g_ref[...] + bln_ref[...]
    out_ref[...] = x_ref[...] + x_new


def _node_mlp(x, t, agg_e, agg_w, w1b, w1c, b1, w2, b2, gg, bln):
    wspec = pl.BlockSpec((H, H), lambda i: (0, 0))
    bspec = pl.BlockSpec((1, H), lambda i: (0, 0))
    agg0 = pl.BlockSpec((1, BN, H), lambda i: (0, i, 0))
    agg1 = pl.BlockSpec((1, BN, H), lambda i: (1, i, 0))
    return pl.pallas_call(
        _node_mlp_body,
        grid=(N // BN,),
        in_specs=[
            pl.BlockSpec((BN, H), lambda i: (i, 0)),
            pl.BlockSpec((1, BN, H), lambda i: (4, i, 0)),
            agg0, agg1, agg0, agg1,
            wspec, wspec, bspec, wspec, bspec, bspec, bspec,
        ],
        out_specs=pl.BlockSpec((BN, H), lambda i: (i, 0)),
        out_shape=jax.ShapeDtypeStruct((N, H), jnp.float32),
    )(x, t, agg_e, agg_e, agg_w, agg_w, w1b, w1c, b1.reshape(1, H), w2,
      b2.reshape(1, H), gg.reshape(1, H), bln.reshape(1, H))


def kernel(x, edge_index, edge_attr, edge_world_index, edge_world_attr,
           emb_W1, emb_b1, emb_W2, emb_b2, emb_g, emb_bln,
           ewb_W1, ewb_b1, ewb_W2, ewb_b2, ewb_g, ewb_bln,
           nb_W1, nb_b1, nb_W2, nb_b2, nb_g, nb_bln):
    src, dst = edge_index[0], edge_index[1]
    wsrc, wdst = edge_world_index[0], edge_world_index[1]

    wcat = jnp.concatenate([
        emb_W1[:H], emb_W1[H:2 * H],
        ewb_W1[:H], ewb_W1[H:2 * H],
        nb_W1[:H],
    ], axis=1)

    t = _proj(x, wcat)
    tf = t.reshape(5 * N, H)

    nm = ((2 * E + GCH - 1) // GCH) * GCH
    nw = ((2 * EW + GCH - 1) // GCH) * GCH
    idx_m = jnp.concatenate([src, dst + N,
                             jnp.zeros((nm - 2 * E,), jnp.int32)])
    idx_w = jnp.concatenate([wsrc + 2 * N, wdst + 3 * N,
                             jnp.zeros((nw - 2 * EW,), jnp.int32)])

    g_m = _sc_gather(tf, idx_m.reshape(nm // GCH, GCH), nm)
    g_w = _sc_gather(tf, idx_w.reshape(nw // GCH, GCH), nw)

    e_new, e_out = _edge_mlp(g_m, edge_attr, 0, E,
                             emb_W1[2 * H:], emb_b1, emb_W2, emb_b2,
                             emb_g, emb_bln, E)
    ew_new, ew_out = _edge_mlp(g_w, edge_world_attr, 0, EW,
                               ewb_W1[2 * H:], ewb_b1, ewb_W2, ewb_b2,
                               ewb_g, ewb_bln, EW)

    zeros = jnp.zeros((NP, H), jnp.float32)
    agg_e = _sc_segment_sum(e_new, dst.reshape(E // SCH, SCH), zeros)
    agg_w = _sc_segment_sum(ew_new, wdst.reshape(EW // SCH, SCH), zeros)

    x_out = _node_mlp(x, t, agg_e, agg_w,
                      nb_W1[H:2 * H], nb_W1[2 * H:], nb_b1, nb_W2, nb_b2,
                      nb_g, nb_bln)
    return (x_out, e_out, ew_out)


# speedup vs baseline: 1.0957x; 1.0957x over previous
import functools

import jax
import jax.numpy as jnp
from jax import lax
from jax.experimental import pallas as pl
from jax.experimental.pallas import tpu as pltpu
from jax.experimental.pallas import tpu_sc as plsc

H = 128
N = 10000
E = 160000
EW = 80000

BN = 1000
BE = 2000

_SC_INFO = plsc.get_sparse_core_info()
NC = _SC_INFO.num_cores
NS = _SC_INFO.num_subcores
NW = NC * NS
GCH = 512
SCH = 128

NP = 10240
_RPT = NP // 16


def _sc_gather(table, idx2d, n_rows):
    nchunks = n_rows // GCH
    mesh = plsc.VectorSubcoreMesh(core_axis_name="c", subcore_axis_name="s")

    @functools.partial(
        pl.kernel, mesh=mesh,
        out_type=jax.ShapeDtypeStruct((n_rows, H), jnp.float32),
        scratch_types=[
            pltpu.VMEM((GCH,), jnp.int32),
            pltpu.VMEM((GCH, H), jnp.float32),
            pltpu.SemaphoreType.DMA,
        ],
    )
    def k(table_hbm, idx_hbm, out_hbm, idx_v, rows_v, sem):
        wid = lax.axis_index("s") * NC + lax.axis_index("c")
        nt = (nchunks - wid + NW - 1) // NW

        def body(i, _):
            j = wid + i * NW
            pltpu.sync_copy(idx_hbm.at[j], idx_v)
            pltpu.async_copy(table_hbm.at[idx_v], rows_v, sem).wait()
            pltpu.sync_copy(rows_v, out_hbm.at[pl.ds(j * GCH, GCH)])
            return 0

        lax.fori_loop(0, nt, body, 0)

    return k(table, idx2d)


def _sc_segment_sum(attr, idx2d, zeros):
    nchunks = idx2d.shape[0]
    mesh = plsc.VectorSubcoreMesh(core_axis_name="c", subcore_axis_name="s")

    @functools.partial(
        pl.kernel, mesh=mesh,
        out_type=jax.ShapeDtypeStruct((NC, NP, H), jnp.float32),
        scratch_types=[
            pltpu.VMEM_SHARED((NP, H), jnp.float32),
            pltpu.VMEM((SCH,), jnp.int32),
            pltpu.VMEM((SCH, H), jnp.float32),
        ],
    )
    def k(attr_hbm, idx_hbm, zeros_hbm, out_hbm, acc, idx_v, rows_v):
        c = lax.axis_index("c")
        s = lax.axis_index("s")
        wid = s * NC + c
        pltpu.sync_copy(zeros_hbm.at[pl.ds(s * _RPT, _RPT)],
                        acc.at[pl.ds(s * _RPT, _RPT)])
        plsc.subcore_barrier()
        nt = (nchunks - wid + NW - 1) // NW

        def body(i, _):
            j = wid + i * NW
            pltpu.sync_copy(idx_hbm.at[j], idx_v)
            pltpu.sync_copy(attr_hbm.at[pl.ds(j * SCH, SCH)], rows_v)
            pltpu.sync_copy(rows_v, acc.at[idx_v], add=True)
            return 0

        lax.fori_loop(0, nt, body, 0)
        plsc.subcore_barrier()
        for kk in range(_RPT // SCH):
            off = s * _RPT + kk * SCH
            pltpu.sync_copy(acc.at[pl.ds(off, SCH)], rows_v)
            pltpu.sync_copy(rows_v, out_hbm.at[c, pl.ds(off, SCH)])

    return k(attr, idx2d, zeros)


def _proj_body(x_ref, w_ref, out_ref):
    xb = x_ref[...]
    for k in range(5):
        out_ref[k] = jnp.dot(xb, w_ref[:, k * H:(k + 1) * H],
                             preferred_element_type=jnp.float32)


def _proj(x, wcat):
    return pl.pallas_call(
        _proj_body,
        grid=(N // BN,),
        in_specs=[
            pl.BlockSpec((BN, H), lambda i: (i, 0)),
            pl.BlockSpec((H, 5 * H), lambda i: (0, 0)),
        ],
        out_specs=pl.BlockSpec((5, BN, H), lambda i: (0, i, 0)),
        out_shape=jax.ShapeDtypeStruct((5, N, H), jnp.float32),
    )(x, wcat)


def _edge_mlp_body(gs_ref, gd_ref, ea_ref, w1c_ref, b1_ref, w2_ref, b2_ref,
                   g_ref, bln_ref, enew_ref, eout_ref):
    ea = ea_ref[...]
    pre = (gs_ref[...] + gd_ref[...] + b1_ref[...]
           + jnp.dot(ea, w1c_ref[...], preferred_element_type=jnp.float32))
    h = jnp.maximum(pre, 0.0)
    z = jnp.dot(h, w2_ref[...], preferred_element_type=jnp.float32) + b2_ref[...]
    mu = jnp.mean(z, axis=-1, keepdims=True)
    var = jnp.mean((z - mu) ** 2, axis=-1, keepdims=True)
    e_new = (z - mu) * jax.lax.rsqrt(var + 1e-5) * g_ref[...] + bln_ref[...]
    enew_ref[...] = e_new
    eout_ref[...] = ea + e_new


def _edge_mlp(g_all, attr, row0_src, row0_dst, w1c, b1, w2, b2, gg, bln, n_rows):
    wspec = pl.BlockSpec((H, H), lambda i: (0, 0))
    bspec = pl.BlockSpec((1, H), lambda i: (0, 0))
    return pl.pallas_call(
        _edge_mlp_body,
        grid=(n_rows // BE,),
        in_specs=[
            pl.BlockSpec((BE, H), lambda i, r=row0_src // BE: (r + i, 0)),
            pl.BlockSpec((BE, H), lambda i, r=row0_dst // BE: (r + i, 0)),
            pl.BlockSpec((BE, H), lambda i: (i, 0)),
            wspec, bspec, wspec, bspec, bspec, bspec,
        ],
        out_specs=[
            pl.BlockSpec((BE, H), lambda i: (i, 0)),
            pl.BlockSpec((BE, H), lambda i: (i, 0)),
        ],
        out_shape=[
            jax.ShapeDtypeStruct((n_rows, H), jnp.float32),
            jax.ShapeDtypeStruct((n_rows, H), jnp.float32),
        ],
    )(g_all, g_all, attr, w1c, b1.reshape(1, H), w2, b2.reshape(1, H),
      gg.reshape(1, H), bln.reshape(1, H))


def _node_mlp_body(x_ref, t_ref, ae0_ref, ae1_ref, aw0_ref, aw1_ref,
                   w1b_ref, w1c_ref, b1_ref, w2_ref, b2_ref, g_ref, bln_ref,
                   out_ref):
    ae = ae0_ref[0] + ae1_ref[0]
    aw = aw0_ref[0] + aw1_ref[0]
    pre = (t_ref[0] + b1_ref[...]
           + jnp.dot(ae, w1b_ref[...], preferred_element_type=jnp.float32)
           + jnp.dot(aw, w1c_ref[...], preferred_element_type=jnp.float32))
    h = jnp.maximum(pre, 0.0)
    z = jnp.dot(h, w2_ref[...], preferred_element_type=jnp.float32) + b2_ref[...]
    mu = jnp.mean(z, axis=-1, keepdims=True)
    var = jnp.mean((z - mu) ** 2, axis=-1, keepdims=True)
    x_new = (z - mu) * jax.lax.rsqrt(var + 1e-5) * g_ref[...] + bln_ref[...]
    out_ref[...] = x_ref[...] + x_new


def _node_mlp(x, t, agg_e, agg_w, w1b, w1c, b1, w2, b2, gg, bln):
    wspec = pl.BlockSpec((H, H), lambda i: (0, 0))
    bspec = pl.BlockSpec((1, H), lambda i: (0, 0))
    agg0 = pl.BlockSpec((1, BN, H), lambda i: (0, i, 0))
    agg1 = pl.BlockSpec((1, BN, H), lambda i: (1, i, 0))
    return pl.pallas_call(
        _node_mlp_body,
        grid=(N // BN,),
        in_specs=[
            pl.BlockSpec((BN, H), lambda i: (i, 0)),
            pl.BlockSpec((1, BN, H), lambda i: (4, i, 0)),
            agg0, agg1, agg0, agg1,
            wspec, wspec, bspec, wspec, bspec, bspec, bspec,
        ],
        out_specs=pl.BlockSpec((BN, H), lambda i: (i, 0)),
        out_shape=jax.ShapeDtypeStruct((N, H), jnp.float32),
    )(x, t, agg_e, agg_e, agg_w, agg_w, w1b, w1c, b1.reshape(1, H), w2,
      b2.reshape(1, H), gg.reshape(1, H), bln.reshape(1, H))


def kernel(x, edge_index, edge_attr, edge_world_index, edge_world_attr,
           emb_W1, emb_b1, emb_W2, emb_b2, emb_g, emb_bln,
           ewb_W1, ewb_b1, ewb_W2, ewb_b2, ewb_g, ewb_bln,
           nb_W1, nb_b1, nb_W2, nb_b2, nb_g, nb_bln):
    src, dst = edge_index[0], edge_index[1]
    wsrc, wdst = edge_world_index[0], edge_world_index[1]

    wcat = jnp.concatenate([
        emb_W1[:H], emb_W1[H:2 * H],
        ewb_W1[:H], ewb_W1[H:2 * H],
        nb_W1[:H],
    ], axis=1)

    t = _proj(x, wcat)
    tf = t.reshape(5 * N, H)

    nm = ((2 * E + GCH - 1) // GCH) * GCH
    nw = ((2 * EW + GCH - 1) // GCH) * GCH
    idx_m = jnp.concatenate([src, dst + N,
                             jnp.zeros((nm - 2 * E,), jnp.int32)])
    idx_w = jnp.concatenate([wsrc + 2 * N, wdst + 3 * N,
                             jnp.zeros((nw - 2 * EW,), jnp.int32)])

    g_m = _sc_gather(tf, idx_m.reshape(nm // GCH, GCH), nm)
    g_w = _sc_gather(tf, idx_w.reshape(nw // GCH, GCH), nw)

    e_new, e_out = _edge_mlp(g_m, edge_attr, 0, E,
                             emb_W1[2 * H:], emb_b1, emb_W2, emb_b2,
                             emb_g, emb_bln, E)
    ew_new, ew_out = _edge_mlp(g_w, edge_world_attr, 0, EW,
                               ewb_W1[2 * H:], ewb_b1, ewb_W2, ewb_b2,
                               ewb_g, ewb_bln, EW)

    zeros = jnp.zeros((NP, H), jnp.float32)
    agg_e = _sc_segment_sum(e_new, dst.reshape(E // SCH, SCH), zeros)
    agg_w = _sc_segment_sum(ew_new, wdst.reshape(EW // SCH, SCH), zeros)

    x_out = _node_mlp(x, t, agg_e, agg_w,
                      nb_W1[H:2 * H], nb_W1[2 * H:], nb_b1, nb_W2, nb_b2,
                      nb_g, nb_bln)
    return (x_out, e_out, ew_out)


# speedup vs baseline: 1.1189x; 1.0212x over previous
import functools

import jax
import jax.numpy as jnp
from jax import lax
from jax.experimental import pallas as pl
from jax.experimental.pallas import tpu as pltpu
from jax.experimental.pallas import tpu_sc as plsc

H = 128
N = 10000
E = 160000
EW = 80000

BN = 2000
BE = 4000

_SC_INFO = plsc.get_sparse_core_info()
NC = _SC_INFO.num_cores
NS = _SC_INFO.num_subcores
NW = NC * NS
GCH = 512
SCH = 128

NP = 10240
_RPT = NP // 16


def _sc_gather(table, idx2d, n_rows):
    nchunks = n_rows // GCH
    mesh = plsc.VectorSubcoreMesh(core_axis_name="c", subcore_axis_name="s")

    @functools.partial(
        pl.kernel, mesh=mesh,
        out_type=jax.ShapeDtypeStruct((n_rows, H), jnp.float32),
        scratch_types=[
            pltpu.VMEM((GCH,), jnp.int32),
            pltpu.VMEM((GCH, H), jnp.float32),
            pltpu.SemaphoreType.DMA,
        ],
    )
    def k(table_hbm, idx_hbm, out_hbm, idx_v, rows_v, sem):
        wid = lax.axis_index("s") * NC + lax.axis_index("c")
        nt = (nchunks - wid + NW - 1) // NW

        def body(i, _):
            j = wid + i * NW
            pltpu.sync_copy(idx_hbm.at[j], idx_v)
            pltpu.async_copy(table_hbm.at[idx_v], rows_v, sem).wait()
            pltpu.sync_copy(rows_v, out_hbm.at[pl.ds(j * GCH, GCH)])
            return 0

        lax.fori_loop(0, nt, body, 0)

    return k(table, idx2d)


def _sc_segment_sum(attr, idx2d, zeros):
    nchunks = idx2d.shape[0]
    mesh = plsc.VectorSubcoreMesh(core_axis_name="c", subcore_axis_name="s")

    @functools.partial(
        pl.kernel, mesh=mesh,
        out_type=jax.ShapeDtypeStruct((NC, NP, H), jnp.float32),
        scratch_types=[
            pltpu.VMEM_SHARED((NP, H), jnp.float32),
            pltpu.VMEM((SCH,), jnp.int32),
            pltpu.VMEM((SCH, H), jnp.float32),
        ],
    )
    def k(attr_hbm, idx_hbm, zeros_hbm, out_hbm, acc, idx_v, rows_v):
        c = lax.axis_index("c")
        s = lax.axis_index("s")
        wid = s * NC + c
        pltpu.sync_copy(zeros_hbm.at[pl.ds(s * _RPT, _RPT)],
                        acc.at[pl.ds(s * _RPT, _RPT)])
        plsc.subcore_barrier()
        nt = (nchunks - wid + NW - 1) // NW

        def body(i, _):
            j = wid + i * NW
            pltpu.sync_copy(idx_hbm.at[j], idx_v)
            pltpu.sync_copy(attr_hbm.at[pl.ds(j * SCH, SCH)], rows_v)
            pltpu.sync_copy(rows_v, acc.at[idx_v], add=True)
            return 0

        lax.fori_loop(0, nt, body, 0)
        plsc.subcore_barrier()
        for kk in range(_RPT // SCH):
            off = s * _RPT + kk * SCH
            pltpu.sync_copy(acc.at[pl.ds(off, SCH)], rows_v)
            pltpu.sync_copy(rows_v, out_hbm.at[c, pl.ds(off, SCH)])

    return k(attr, idx2d, zeros)


def _proj_body(x_ref, w_ref, out_ref):
    xb = x_ref[...]
    for k in range(5):
        out_ref[k] = jnp.dot(xb, w_ref[:, k * H:(k + 1) * H],
                             preferred_element_type=jnp.float32)


def _proj(x, wcat):
    return pl.pallas_call(
        _proj_body,
        grid=(N // BN,),
        in_specs=[
            pl.BlockSpec((BN, H), lambda i: (i, 0)),
            pl.BlockSpec((H, 5 * H), lambda i: (0, 0)),
        ],
        out_specs=pl.BlockSpec((5, BN, H), lambda i: (0, i, 0)),
        out_shape=jax.ShapeDtypeStruct((5, N, H), jnp.float32),
    )(x, wcat)


def _edge_mlp_body(gs_ref, gd_ref, ea_ref, w1c_ref, b1_ref, w2_ref, b2_ref,
                   g_ref, bln_ref, enew_ref, eout_ref):
    ea = ea_ref[...]
    pre = (gs_ref[...] + gd_ref[...] + b1_ref[...]
           + jnp.dot(ea, w1c_ref[...], preferred_element_type=jnp.float32))
    h = jnp.maximum(pre, 0.0)
    z = jnp.dot(h, w2_ref[...], preferred_element_type=jnp.float32) + b2_ref[...]
    mu = jnp.mean(z, axis=-1, keepdims=True)
    var = jnp.mean((z - mu) ** 2, axis=-1, keepdims=True)
    e_new = (z - mu) * jax.lax.rsqrt(var + 1e-5) * g_ref[...] + bln_ref[...]
    enew_ref[...] = e_new
    eout_ref[...] = ea + e_new


def _edge_mlp(g_all, attr, row0_src, row0_dst, w1c, b1, w2, b2, gg, bln, n_rows):
    wspec = pl.BlockSpec((H, H), lambda i: (0, 0))
    bspec = pl.BlockSpec((1, H), lambda i: (0, 0))
    return pl.pallas_call(
        _edge_mlp_body,
        grid=(n_rows // BE,),
        in_specs=[
            pl.BlockSpec((BE, H), lambda i, r=row0_src // BE: (r + i, 0)),
            pl.BlockSpec((BE, H), lambda i, r=row0_dst // BE: (r + i, 0)),
            pl.BlockSpec((BE, H), lambda i: (i, 0)),
            wspec, bspec, wspec, bspec, bspec, bspec,
        ],
        out_specs=[
            pl.BlockSpec((BE, H), lambda i: (i, 0)),
            pl.BlockSpec((BE, H), lambda i: (i, 0)),
        ],
        out_shape=[
            jax.ShapeDtypeStruct((n_rows, H), jnp.float32),
            jax.ShapeDtypeStruct((n_rows, H), jnp.float32),
        ],
    )(g_all, g_all, attr, w1c, b1.reshape(1, H), w2, b2.reshape(1, H),
      gg.reshape(1, H), bln.reshape(1, H))


def _node_mlp_body(x_ref, t_ref, ae0_ref, ae1_ref, aw0_ref, aw1_ref,
                   w1b_ref, w1c_ref, b1_ref, w2_ref, b2_ref, g_ref, bln_ref,
                   out_ref):
    ae = ae0_ref[0] + ae1_ref[0]
    aw = aw0_ref[0] + aw1_ref[0]
    pre = (t_ref[0] + b1_ref[...]
           + jnp.dot(ae, w1b_ref[...], preferred_element_type=jnp.float32)
           + jnp.dot(aw, w1c_ref[...], preferred_element_type=jnp.float32))
    h = jnp.maximum(pre, 0.0)
    z = jnp.dot(h, w2_ref[...], preferred_element_type=jnp.float32) + b2_ref[...]
    mu = jnp.mean(z, axis=-1, keepdims=True)
    var = jnp.mean((z - mu) ** 2, axis=-1, keepdims=True)
    x_new = (z - mu) * jax.lax.rsqrt(var + 1e-5) * g_ref[...] + bln_ref[...]
    out_ref[...] = x_ref[...] + x_new


def _node_mlp(x, t, agg_e, agg_w, w1b, w1c, b1, w2, b2, gg, bln):
    wspec = pl.BlockSpec((H, H), lambda i: (0, 0))
    bspec = pl.BlockSpec((1, H), lambda i: (0, 0))
    agg0 = pl.BlockSpec((1, BN, H), lambda i: (0, i, 0))
    agg1 = pl.BlockSpec((1, BN, H), lambda i: (1, i, 0))
    return pl.pallas_call(
        _node_mlp_body,
        grid=(N // BN,),
        in_specs=[
            pl.BlockSpec((BN, H), lambda i: (i, 0)),
            pl.BlockSpec((1, BN, H), lambda i: (4, i, 0)),
            agg0, agg1, agg0, agg1,
            wspec, wspec, bspec, wspec, bspec, bspec, bspec,
        ],
        out_specs=pl.BlockSpec((BN, H), lambda i: (i, 0)),
        out_shape=jax.ShapeDtypeStruct((N, H), jnp.float32),
    )(x, t, agg_e, agg_e, agg_w, agg_w, w1b, w1c, b1.reshape(1, H), w2,
      b2.reshape(1, H), gg.reshape(1, H), bln.reshape(1, H))


def kernel(x, edge_index, edge_attr, edge_world_index, edge_world_attr,
           emb_W1, emb_b1, emb_W2, emb_b2, emb_g, emb_bln,
           ewb_W1, ewb_b1, ewb_W2, ewb_b2, ewb_g, ewb_bln,
           nb_W1, nb_b1, nb_W2, nb_b2, nb_g, nb_bln):
    src, dst = edge_index[0], edge_index[1]
    wsrc, wdst = edge_world_index[0], edge_world_index[1]

    wcat = jnp.concatenate([
        emb_W1[:H], emb_W1[H:2 * H],
        ewb_W1[:H], ewb_W1[H:2 * H],
        nb_W1[:H],
    ], axis=1)

    t = _proj(x, wcat)
    tf = t.reshape(5 * N, H)

    nm = ((2 * E + GCH - 1) // GCH) * GCH
    nw = ((2 * EW + GCH - 1) // GCH) * GCH
    idx_m = jnp.concatenate([src, dst + N,
                             jnp.zeros((nm - 2 * E,), jnp.int32)])
    idx_w = jnp.concatenate([wsrc + 2 * N, wdst + 3 * N,
                             jnp.zeros((nw - 2 * EW,), jnp.int32)])

    g_m = _sc_gather(tf, idx_m.reshape(nm // GCH, GCH), nm)
    g_w = _sc_gather(tf, idx_w.reshape(nw // GCH, GCH), nw)

    e_new, e_out = _edge_mlp(g_m, edge_attr, 0, E,
                             emb_W1[2 * H:], emb_b1, emb_W2, emb_b2,
                             emb_g, emb_bln, E)
    ew_new, ew_out = _edge_mlp(g_w, edge_world_attr, 0, EW,
                               ewb_W1[2 * H:], ewb_b1, ewb_W2, ewb_b2,
                               ewb_g, ewb_bln, EW)

    zeros = jnp.zeros((NP, H), jnp.float32)
    agg_e = _sc_segment_sum(e_new, dst.reshape(E // SCH, SCH), zeros)
    agg_w = _sc_segment_sum(ew_new, wdst.reshape(EW // SCH, SCH), zeros)

    x_out = _node_mlp(x, t, agg_e, agg_w,
                      nb_W1[H:2 * H], nb_W1[2 * H:], nb_b1, nb_W2, nb_b2,
                      nb_g, nb_bln)
    return (x_out, e_out, ew_out)


# speedup vs baseline: 1.1261x; 1.0064x over previous
import functools

import jax
import jax.numpy as jnp
from jax import lax
from jax.experimental import pallas as pl
from jax.experimental.pallas import tpu as pltpu
from jax.experimental.pallas import tpu_sc as plsc

H = 128
N = 10000
E = 160000
EW = 80000

BN = 2000
BE = 8000

_SC_INFO = plsc.get_sparse_core_info()
NC = _SC_INFO.num_cores
NS = _SC_INFO.num_subcores
NW = NC * NS
GCH = 512
SCH = 128

NP = 10240
_RPT = NP // 16


def _sc_gather(table, idx2d, n_rows):
    nchunks = n_rows // GCH
    mesh = plsc.VectorSubcoreMesh(core_axis_name="c", subcore_axis_name="s")

    @functools.partial(
        pl.kernel, mesh=mesh,
        out_type=jax.ShapeDtypeStruct((n_rows, H), jnp.float32),
        scratch_types=[
            pltpu.VMEM((GCH,), jnp.int32),
            pltpu.VMEM((GCH, H), jnp.float32),
            pltpu.SemaphoreType.DMA,
        ],
    )
    def k(table_hbm, idx_hbm, out_hbm, idx_v, rows_v, sem):
        wid = lax.axis_index("s") * NC + lax.axis_index("c")
        nt = (nchunks - wid + NW - 1) // NW

        def body(i, _):
            j = wid + i * NW
            pltpu.sync_copy(idx_hbm.at[j], idx_v)
            pltpu.async_copy(table_hbm.at[idx_v], rows_v, sem).wait()
            pltpu.sync_copy(rows_v, out_hbm.at[pl.ds(j * GCH, GCH)])
            return 0

        lax.fori_loop(0, nt, body, 0)

    return k(table, idx2d)


def _sc_segment_sum(attr, idx2d, zeros):
    nchunks = idx2d.shape[0]
    mesh = plsc.VectorSubcoreMesh(core_axis_name="c", subcore_axis_name="s")

    @functools.partial(
        pl.kernel, mesh=mesh,
        out_type=jax.ShapeDtypeStruct((NC, NP, H), jnp.float32),
        scratch_types=[
            pltpu.VMEM_SHARED((NP, H), jnp.float32),
            pltpu.VMEM((SCH,), jnp.int32),
            pltpu.VMEM((SCH, H), jnp.float32),
        ],
    )
    def k(attr_hbm, idx_hbm, zeros_hbm, out_hbm, acc, idx_v, rows_v):
        c = lax.axis_index("c")
        s = lax.axis_index("s")
        wid = s * NC + c
        pltpu.sync_copy(zeros_hbm.at[pl.ds(s * _RPT, _RPT)],
                        acc.at[pl.ds(s * _RPT, _RPT)])
        plsc.subcore_barrier()
        nt = (nchunks - wid + NW - 1) // NW

        def body(i, _):
            j = wid + i * NW
            pltpu.sync_copy(idx_hbm.at[j], idx_v)
            pltpu.sync_copy(attr_hbm.at[pl.ds(j * SCH, SCH)], rows_v)
            pltpu.sync_copy(rows_v, acc.at[idx_v], add=True)
            return 0

        lax.fori_loop(0, nt, body, 0)
        plsc.subcore_barrier()
        for kk in range(_RPT // SCH):
            off = s * _RPT + kk * SCH
            pltpu.sync_copy(acc.at[pl.ds(off, SCH)], rows_v)
            pltpu.sync_copy(rows_v, out_hbm.at[c, pl.ds(off, SCH)])

    return k(attr, idx2d, zeros)


def _proj_body(x_ref, w_ref, out_ref):
    xb = x_ref[...]
    for k in range(5):
        out_ref[k] = jnp.dot(xb, w_ref[:, k * H:(k + 1) * H],
                             preferred_element_type=jnp.float32)


def _proj(x, wcat):
    return pl.pallas_call(
        _proj_body,
        grid=(N // BN,),
        in_specs=[
            pl.BlockSpec((BN, H), lambda i: (i, 0)),
            pl.BlockSpec((H, 5 * H), lambda i: (0, 0)),
        ],
        out_specs=pl.BlockSpec((5, BN, H), lambda i: (0, i, 0)),
        out_shape=jax.ShapeDtypeStruct((5, N, H), jnp.float32),
    )(x, wcat)


def _edge_mlp_body(gs_ref, gd_ref, ea_ref, w1c_ref, b1_ref, w2_ref, b2_ref,
                   g_ref, bln_ref, enew_ref, eout_ref):
    ea = ea_ref[...]
    pre = (gs_ref[...] + gd_ref[...] + b1_ref[...]
           + jnp.dot(ea, w1c_ref[...], preferred_element_type=jnp.float32))
    h = jnp.maximum(pre, 0.0)
    z = jnp.dot(h, w2_ref[...], preferred_element_type=jnp.float32) + b2_ref[...]
    mu = jnp.mean(z, axis=-1, keepdims=True)
    var = jnp.mean((z - mu) ** 2, axis=-1, keepdims=True)
    e_new = (z - mu) * jax.lax.rsqrt(var + 1e-5) * g_ref[...] + bln_ref[...]
    enew_ref[...] = e_new
    eout_ref[...] = ea + e_new


def _edge_mlp(g_all, attr, row0_src, row0_dst, w1c, b1, w2, b2, gg, bln, n_rows):
    wspec = pl.BlockSpec((H, H), lambda i: (0, 0))
    bspec = pl.BlockSpec((1, H), lambda i: (0, 0))
    return pl.pallas_call(
        _edge_mlp_body,
        grid=(n_rows // BE,),
        in_specs=[
            pl.BlockSpec((BE, H), lambda i, r=row0_src // BE: (r + i, 0)),
            pl.BlockSpec((BE, H), lambda i, r=row0_dst // BE: (r + i, 0)),
            pl.BlockSpec((BE, H), lambda i: (i, 0)),
            wspec, bspec, wspec, bspec, bspec, bspec,
        ],
        out_specs=[
            pl.BlockSpec((BE, H), lambda i: (i, 0)),
            pl.BlockSpec((BE, H), lambda i: (i, 0)),
        ],
        out_shape=[
            jax.ShapeDtypeStruct((n_rows, H), jnp.float32),
            jax.ShapeDtypeStruct((n_rows, H), jnp.float32),
        ],
    )(g_all, g_all, attr, w1c, b1.reshape(1, H), w2, b2.reshape(1, H),
      gg.reshape(1, H), bln.reshape(1, H))


def _node_mlp_body(x_ref, t_ref, ae0_ref, ae1_ref, aw0_ref, aw1_ref,
                   w1b_ref, w1c_ref, b1_ref, w2_ref, b2_ref, g_ref, bln_ref,
                   out_ref):
    ae = ae0_ref[0] + ae1_ref[0]
    aw = aw0_ref[0] + aw1_ref[0]
    pre = (t_ref[0] + b1_ref[...]
           + jnp.dot(ae, w1b_ref[...], preferred_element_type=jnp.float32)
           + jnp.dot(aw, w1c_ref[...], preferred_element_type=jnp.float32))
    h = jnp.maximum(pre, 0.0)
    z = jnp.dot(h, w2_ref[...], preferred_element_type=jnp.float32) + b2_ref[...]
    mu = jnp.mean(z, axis=-1, keepdims=True)
    var = jnp.mean((z - mu) ** 2, axis=-1, keepdims=True)
    x_new = (z - mu) * jax.lax.rsqrt(var + 1e-5) * g_ref[...] + bln_ref[...]
    out_ref[...] = x_ref[...] + x_new


def _node_mlp(x, t, agg_e, agg_w, w1b, w1c, b1, w2, b2, gg, bln):
    wspec = pl.BlockSpec((H, H), lambda i: (0, 0))
    bspec = pl.BlockSpec((1, H), lambda i: (0, 0))
    agg0 = pl.BlockSpec((1, BN, H), lambda i: (0, i, 0))
    agg1 = pl.BlockSpec((1, BN, H), lambda i: (1, i, 0))
    return pl.pallas_call(
        _node_mlp_body,
        grid=(N // BN,),
        in_specs=[
            pl.BlockSpec((BN, H), lambda i: (i, 0)),
            pl.BlockSpec((1, BN, H), lambda i: (4, i, 0)),
            agg0, agg1, agg0, agg1,
            wspec, wspec, bspec, wspec, bspec, bspec, bspec,
        ],
        out_specs=pl.BlockSpec((BN, H), lambda i: (i, 0)),
        out_shape=jax.ShapeDtypeStruct((N, H), jnp.float32),
    )(x, t, agg_e, agg_e, agg_w, agg_w, w1b, w1c, b1.reshape(1, H), w2,
      b2.reshape(1, H), gg.reshape(1, H), bln.reshape(1, H))


def kernel(x, edge_index, edge_attr, edge_world_index, edge_world_attr,
           emb_W1, emb_b1, emb_W2, emb_b2, emb_g, emb_bln,
           ewb_W1, ewb_b1, ewb_W2, ewb_b2, ewb_g, ewb_bln,
           nb_W1, nb_b1, nb_W2, nb_b2, nb_g, nb_bln):
    src, dst = edge_index[0], edge_index[1]
    wsrc, wdst = edge_world_index[0], edge_world_index[1]

    wcat = jnp.concatenate([
        emb_W1[:H], emb_W1[H:2 * H],
        ewb_W1[:H], ewb_W1[H:2 * H],
        nb_W1[:H],
    ], axis=1)

    t = _proj(x, wcat)
    tf = t.reshape(5 * N, H)

    nm = ((2 * E + GCH - 1) // GCH) * GCH
    nw = ((2 * EW + GCH - 1) // GCH) * GCH
    idx_m = jnp.concatenate([src, dst + N,
                             jnp.zeros((nm - 2 * E,), jnp.int32)])
    idx_w = jnp.concatenate([wsrc + 2 * N, wdst + 3 * N,
                             jnp.zeros((nw - 2 * EW,), jnp.int32)])

    g_m = _sc_gather(tf, idx_m.reshape(nm // GCH, GCH), nm)
    g_w = _sc_gather(tf, idx_w.reshape(nw // GCH, GCH), nw)

    e_new, e_out = _edge_mlp(g_m, edge_attr, 0, E,
                             emb_W1[2 * H:], emb_b1, emb_W2, emb_b2,
                             emb_g, emb_bln, E)
    ew_new, ew_out = _edge_mlp(g_w, edge_world_attr, 0, EW,
                               ewb_W1[2 * H:], ewb_b1, ewb_W2, ewb_b2,
                               ewb_g, ewb_bln, EW)

    zeros = jnp.zeros((NP, H), jnp.float32)
    agg_e = _sc_segment_sum(e_new, dst.reshape(E // SCH, SCH), zeros)
    agg_w = _sc_segment_sum(ew_new, wdst.reshape(EW // SCH, SCH), zeros)

    x_out = _node_mlp(x, t, agg_e, agg_w,
                      nb_W1[H:2 * H], nb_W1[2 * H:], nb_b1, nb_W2, nb_b2,
                      nb_g, nb_bln)
    return (x_out, e_out, ew_out)
